# Initial kernel scaffold; baseline (speedup 1.0000x reference)
import jax
import jax.numpy as jnp
from jax.experimental import pallas as pl


def kernel(xc, yc, xt, pos, edge_index, enc_w0, enc_b0, enc_w1, enc_b1, enc_w2, enc_b2, gcn_w, gcn_b, dec_w0, dec_b0, dec_w1, dec_b1, dec_w2, dec_b2):
    raise NotImplementedError("write your pallas kernel here")


# speedup vs baseline: 8.9269x; 8.9269x over previous
import functools

import jax
import jax.numpy as jnp
from jax import lax
from jax.experimental import pallas as pl
from jax.experimental.pallas import tpu as pltpu
from jax.experimental.pallas import tpu_sc as plsc

_N = 10000
_NPAD = 10240
_B = 2
_NC = 1024
_NT = 1024
_DH = 128
_E = 320000
_CB = 128
_RB = 1280

_TILES = 16
_CHUNK = 128
_CT = 157
_PER_TILE = _CT * _CHUNK
_EPC = _PER_TILE * _TILES

_DEGW = 8

_ROWS_PER_TILE = _NPAD // _TILES


@functools.lru_cache(maxsize=None)
def _sc_mesh():
    return plsc.VectorSubcoreMesh(core_axis_name="c", subcore_axis_name="s")


@functools.lru_cache(maxsize=None)
def _build_scatter_sc():
    return functools.partial(
        pl.kernel,
        out_type=jax.ShapeDtypeStruct((_B * _NPAD, _DH), jnp.float32),
        mesh=_sc_mesh(),
        scratch_types=[
            pltpu.VMEM((_CHUNK,), jnp.int32),
            pltpu.VMEM((_CHUNK,), jnp.int32),
            pltpu.VMEM((_CHUNK, _DH), jnp.float32),
            pltpu.VMEM_SHARED((_NPAD, _DH), jnp.float32),
            pltpu.SemaphoreType.DMA,
        ],
    )(_scatter_body)


def _scatter_body(y_hbm, srcg_hbm, dstl_hbm, out_hbm, src_v, dst_v, rows_v, acc_sh, sem):
    c = lax.axis_index("c")
    s = lax.axis_index("s")
    r0 = s * _ROWS_PER_TILE
    pltpu.sync_copy(y_hbm.at[pl.ds(c * _NPAD + r0, _ROWS_PER_TILE)],
                    acc_sh.at[pl.ds(r0, _ROWS_PER_TILE)])
    plsc.subcore_barrier()
    base = s * _PER_TILE

    def body(j, carry):
        off = base + j * _CHUNK
        pltpu.sync_copy(srcg_hbm.at[c, pl.ds(off, _CHUNK)], src_v)
        pltpu.sync_copy(dstl_hbm.at[pl.ds(off, _CHUNK)], dst_v)
        pltpu.async_copy(y_hbm.at[src_v], rows_v, sem).wait()
        pltpu.sync_copy(rows_v, acc_sh.at[dst_v], add=True)
        return carry

    lax.fori_loop(0, _CT, body, 0)
    plsc.subcore_barrier()
    pltpu.sync_copy(acc_sh.at[pl.ds(r0, _ROWS_PER_TILE)],
                    out_hbm.at[pl.ds(c * _NPAD + r0, _ROWS_PER_TILE)])


def _scatter_sc(y, src_g, dst_l):
    return _build_scatter_sc()(y, src_g, dst_l)


def _masked_softmax_scores(x, pt):
    xsq = jnp.sum(x * x, axis=1, keepdims=True)
    psq = jnp.sum(pt * pt, axis=0, keepdims=True)
    logits = 2.0 * jnp.dot(x, pt, preferred_element_type=jnp.float32) - xsq - psq
    col = lax.broadcasted_iota(jnp.int32, logits.shape, 1)
    logits = jnp.where(col < _N, logits, -1e30)
    m = jnp.max(logits, axis=1, keepdims=True)
    p = jnp.exp(logits - m)
    return p / jnp.sum(p, axis=1, keepdims=True)


def _assign_body(xc_ref, yc_ref, pt_ref, w0x_ref, w0y_ref, b0_ref, w1_ref,
                 b1_ref, w2_ref, b2_ref, out_ref):
    cb = pl.program_id(1)
    x = xc_ref[0]
    y = yc_ref[0]
    h = jnp.dot(x, w0x_ref[...], preferred_element_type=jnp.float32) \
        + jnp.dot(y, w0y_ref[...], preferred_element_type=jnp.float32) \
        + b0_ref[...]
    h = jnp.maximum(h, 0.0)
    h = jnp.maximum(jnp.dot(h, w1_ref[...], preferred_element_type=jnp.float32)
                    + b1_ref[...], 0.0)
    emb = jnp.dot(h, w2_ref[...], preferred_element_type=jnp.float32) + b2_ref[...]
    sc = _masked_softmax_scores(x, pt_ref[...])
    contrib = lax.dot_general(sc, emb, (((0,), (0,)), ((), ())),
                              preferred_element_type=jnp.float32)

    @pl.when(cb == 0)
    def _():
        out_ref[...] = jnp.zeros_like(out_ref)

    out_ref[0] += contrib


_assign = pl.pallas_call(
    _assign_body,
    grid=(_B, _NC // _CB),
    in_specs=[
        pl.BlockSpec((1, _CB, 3), lambda b, c: (b, c, 0)),
        pl.BlockSpec((1, _CB, 4), lambda b, c: (b, c, 0)),
        pl.BlockSpec((3, _NPAD), lambda b, c: (0, 0)),
        pl.BlockSpec((3, _DH), lambda b, c: (0, 0)),
        pl.BlockSpec((4, _DH), lambda b, c: (0, 0)),
        pl.BlockSpec((1, _DH), lambda b, c: (0, 0)),
        pl.BlockSpec((_DH, _DH), lambda b, c: (0, 0)),
        pl.BlockSpec((1, _DH), lambda b, c: (0, 0)),
        pl.BlockSpec((_DH, _DH), lambda b, c: (0, 0)),
        pl.BlockSpec((1, _DH), lambda b, c: (0, 0)),
    ],
    out_specs=pl.BlockSpec((1, _NPAD, _DH), lambda b, c: (b, 0, 0)),
    out_shape=jax.ShapeDtypeStruct((_B, _NPAD, _DH), jnp.float32),
)


def _dinv_block(deg_ref):
    return lax.rsqrt(deg_ref[0, :, 0:1])


def _make_step(with_post):
    def body(z_ref, deg_ref, pos_ref, wx_ref, wp_ref, bias_ref, out_ref):
        dinv = _dinv_block(deg_ref)
        x = z_ref[0]
        if with_post:
            x = dinv * x + bias_ref[...]
        y = jnp.dot(x, wx_ref[...], preferred_element_type=jnp.float32) \
            + jnp.dot(pos_ref[...], wp_ref[...], preferred_element_type=jnp.float32)
        out_ref[0] = dinv * y

    return pl.pallas_call(
        body,
        grid=(_B, _NPAD // _RB),
        in_specs=[
            pl.BlockSpec((1, _RB, _DH), lambda b, n: (b, n, 0)),
            pl.BlockSpec((1, _RB, _DEGW), lambda b, n: (0, n, 0)),
            pl.BlockSpec((_RB, 3), lambda b, n: (n, 0)),
            pl.BlockSpec((_DH, _DH), lambda b, n: (0, 0)),
            pl.BlockSpec((3, _DH), lambda b, n: (0, 0)),
            pl.BlockSpec((1, _DH), lambda b, n: (0, 0)),
        ],
        out_specs=pl.BlockSpec((1, _RB, _DH), lambda b, n: (b, n, 0)),
        out_shape=jax.ShapeDtypeStruct((_B, _NPAD, _DH), jnp.float32),
    )


_pre0 = _make_step(False)
_step = _make_step(True)


def _decode_body(xt_ref, z_ref, deg_ref, pt_ref, gb_ref, w0z_ref, w0x_ref,
                 b0_ref, w1_ref, b1_ref, w2_ref, b2_ref, out_ref):
    x = xt_ref[0]
    dinv = _dinv_block(deg_ref)
    lat = dinv * z_ref[0] + gb_ref[...]
    st = _masked_softmax_scores(x, pt_ref[...])
    z = jnp.dot(st, lat, preferred_element_type=jnp.float32)
    h = jnp.dot(z, w0z_ref[...], preferred_element_type=jnp.float32) \
        + jnp.dot(x, w0x_ref[...], preferred_element_type=jnp.float32) \
        + b0_ref[...]
    h = jnp.maximum(h, 0.0)
    h = jnp.maximum(jnp.dot(h, w1_ref[...], preferred_element_type=jnp.float32)
                    + b1_ref[...], 0.0)
    out_ref[0] = jnp.dot(h, w2_ref[...], preferred_element_type=jnp.float32) \
        + b2_ref[...]


_decode = pl.pallas_call(
    _decode_body,
    grid=(_B, _NT // _CB),
    in_specs=[
        pl.BlockSpec((1, _CB, 3), lambda b, t: (b, t, 0)),
        pl.BlockSpec((1, _NPAD, _DH), lambda b, t: (b, 0, 0)),
        pl.BlockSpec((1, _NPAD, _DEGW), lambda b, t: (0, 0, 0)),
        pl.BlockSpec((3, _NPAD), lambda b, t: (0, 0)),
        pl.BlockSpec((1, _DH), lambda b, t: (0, 0)),
        pl.BlockSpec((_DH, _DH), lambda b, t: (0, 0)),
        pl.BlockSpec((3, _DH), lambda b, t: (0, 0)),
        pl.BlockSpec((1, _DH), lambda b, t: (0, 0)),
        pl.BlockSpec((_DH, _DH), lambda b, t: (0, 0)),
        pl.BlockSpec((1, _DH), lambda b, t: (0, 0)),
        pl.BlockSpec((_DH, _DH), lambda b, t: (0, 0)),
        pl.BlockSpec((1, _DH), lambda b, t: (0, 0)),
    ],
    out_specs=pl.BlockSpec((1, _CB, _DH), lambda b, t: (b, t, 0)),
    out_shape=jax.ShapeDtypeStruct((_B, _NT, _DH), jnp.float32),
)


def kernel(xc, yc, xt, pos, edge_index, enc_w0, enc_b0, enc_w1, enc_b1,
           enc_w2, enc_b2, gcn_w, gcn_b, dec_w0, dec_b0, dec_w1, dec_b1,
           dec_w2, dec_b2):
    f32 = jnp.float32
    pos_p = jnp.pad(pos, ((0, _NPAD - _N), (0, 0)))
    pos_t = pos_p.T

    src = edge_index[0].astype(jnp.int32)
    dst = edge_index[1].astype(jnp.int32)
    pad_e = _EPC - _E
    src_pad = jnp.concatenate([src, jnp.zeros((pad_e,), jnp.int32)])
    src_g = src_pad[None, :] + (jnp.arange(_B, dtype=jnp.int32) * _NPAD)[:, None]
    dst_l = jnp.concatenate([dst, jnp.full((pad_e,), _N, jnp.int32)])

    degz = _scatter_sc(jnp.ones((_B * _NPAD, _DH), f32), src_g, dst_l)
    deg = degz[:_NPAD, :_DEGW].reshape(1, _NPAD, _DEGW)

    lat = _assign(xc, yc, pos_t, enc_w0[:3], enc_w0[3:],
                  enc_b0.reshape(1, -1), enc_w1, enc_b1.reshape(1, -1),
                  enc_w2, enc_b2.reshape(1, -1))

    wx = gcn_w[:_DH]
    wp = gcn_w[_DH:]
    gb = gcn_b.reshape(1, -1)
    y = _pre0(lat, deg, pos_p, wx, wp, gb)
    z = None
    for i in range(3):
        z = _scatter_sc(y.reshape(_B * _NPAD, _DH), src_g, dst_l)
        z = z.reshape(_B, _NPAD, _DH)
        if i < 2:
            y = _step(z, deg, pos_p, wx, wp, gb)

    w2p = jnp.pad(dec_w2, ((0, 0), (0, _DH - dec_w2.shape[1])))
    b2p = jnp.pad(dec_b2, (0, _DH - dec_b2.shape[0])).reshape(1, -1)
    out = _decode(xt, z, deg, pos_t, gb, dec_w0[:_DH], dec_w0[_DH:],
                  dec_b0.reshape(1, -1), dec_w1, dec_b1.reshape(1, -1),
                  w2p, b2p)
    return out[:, :, :4]
